# baseline (speedup 1.0000x reference)
import functools

import jax
import jax.numpy as jnp
from jax import lax
from jax.experimental import pallas as pl
from jax.experimental.pallas import tpu as pltpu
from jax.experimental.pallas import tpu_sc as plsc

_N = 10000
_E = 320000
_D = 128
_NC = 2
_NS = 16
_NW = _NC * _NS
_C = 128
_ER = 2560
_EPAD = _ER * _C - _E
_RW = _ER // _NW
_RING = 4
_RC = 104
_RPT = 624
_RTAIL = _N - _NS * _RPT
_DPT = 624
_DTAIL = _N - _NS * _DPT

_mesh = plsc.VectorSubcoreMesh(core_axis_name="c", subcore_axis_name="s")
_f32 = jnp.float32


@functools.partial(
    pl.kernel,
    out_type=jax.ShapeDtypeStruct((_NC * _N,), _f32),
    mesh=_mesh,
    scratch_types=[
        pltpu.VMEM((_RW, _C), jnp.int32),
        pltpu.VMEM((_C,), _f32),
        pltpu.VMEM((_DPT,), _f32),
        pltpu.VMEM_SHARED((_N + 8,), _f32),
        pltpu.SemaphoreType.DMA,
        pltpu.SemaphoreType.DMA,
    ],
)
def _deg_kernel(dst_hbm, out_hbm, dst2d, ones_v, buf_v, acc_sh,
                sem_i, sem_s):
    c = lax.axis_index("c")
    s = lax.axis_index("s")
    wid = c * _NS + s
    pltpu.async_copy(dst_hbm.at[pl.ds(wid * _RW, _RW)], dst2d, sem_i)

    for j in range(_C // 16):
        ones_v[pl.ds(j * 16, 16)] = jnp.ones((16,), _f32)
    for j in range(_DPT // 16):
        buf_v[pl.ds(j * 16, 16)] = jnp.zeros((16,), _f32)
    pltpu.sync_copy(buf_v, acc_sh.at[pl.ds(s * _DPT, _DPT)])

    @pl.when(s == _NS - 1)
    def _():
        pltpu.sync_copy(buf_v.at[pl.ds(0, _DTAIL)],
                        acc_sh.at[pl.ds(_NS * _DPT, _DTAIL)])
        pltpu.sync_copy(buf_v.at[pl.ds(0, 8)],
                        acc_sh.at[pl.ds(_N, 8)])

    pltpu.make_async_copy(dst_hbm.at[pl.ds(wid * _RW, _RW)],
                          dst2d, sem_i).wait()
    plsc.subcore_barrier()

    def body(i, carry):
        pltpu.async_copy(ones_v, acc_sh.at[dst2d.at[i]], sem_s, add=True)
        return carry

    lax.fori_loop(0, _RW, body, 0)

    def drain(i, carry):
        pltpu.make_async_copy(ones_v, acc_sh.at[dst2d.at[0]], sem_s).wait()
        return carry

    lax.fori_loop(0, _RW, drain, 0)
    plsc.subcore_barrier()

    pltpu.sync_copy(acc_sh.at[pl.ds(s * _DPT, _DPT)], buf_v)
    pltpu.sync_copy(buf_v, out_hbm.at[pl.ds(c * _N + s * _DPT, _DPT)])

    @pl.when(s == _NS - 1)
    def _():
        pltpu.sync_copy(acc_sh.at[pl.ds(_NS * _DPT, _DTAIL)],
                        buf_v.at[pl.ds(0, _DTAIL)])
        pltpu.sync_copy(buf_v.at[pl.ds(0, _DTAIL)],
                        out_hbm.at[pl.ds(c * _N + _NS * _DPT, _DTAIL)])


@functools.partial(
    pl.kernel,
    out_type=jax.ShapeDtypeStruct((_NC, _N, _D), _f32),
    mesh=_mesh,
    scratch_types=[
        pltpu.VMEM((_C,), jnp.int32),
        pltpu.VMEM((_C,), jnp.int32),
        pltpu.VMEM((_RW, _C), jnp.int32),
        pltpu.VMEM((_C, _D), _f32),
        pltpu.VMEM((_C, _D), _f32),
        pltpu.VMEM_SHARED((_N + 8, _D), _f32),
        pltpu.SemaphoreType.DMA,
        pltpu.SemaphoreType.DMA,
        pltpu.SemaphoreType.DMA,
        pltpu.SemaphoreType.DMA,
        pltpu.SemaphoreType.DMA,
    ],
)
def _hop_kernel(g_hbm, src_hbm, dst_hbm, out_hbm,
                src_a, src_b, dst2d, rows_a, rows_b, acc_sh,
                sem_d, sem_ia, sem_ib, sem_ga, sem_gb):
    c = lax.axis_index("c")
    s = lax.axis_index("s")
    wid = c * _NS + s
    e0 = wid * _RW * _C
    pltpu.async_copy(dst_hbm.at[pl.ds(wid * _RW, _RW)], dst2d, sem_d)

    def zrow(r, carry):
        for j in range(_D // 16):
            rows_a[r, pl.ds(j * 16, 16)] = jnp.zeros((16,), _f32)
        return carry

    lax.fori_loop(0, _C, zrow, 0)
    r0 = s * _RPT
    for k in range(4):
        pltpu.sync_copy(rows_a, acc_sh.at[pl.ds(r0 + k * _C, _C)])
    pltpu.sync_copy(rows_a.at[pl.ds(0, _RPT - 4 * _C)],
                    acc_sh.at[pl.ds(r0 + 4 * _C, _RPT - 4 * _C)])

    @pl.when(s == _NS - 1)
    def _():
        pltpu.sync_copy(rows_a.at[pl.ds(0, _RTAIL)],
                        acc_sh.at[pl.ds(_NS * _RPT, _RTAIL)])

    pltpu.make_async_copy(dst_hbm.at[pl.ds(wid * _RW, _RW)],
                          dst2d, sem_d).wait()
    plsc.subcore_barrier()

    def ld(i, idx, sem):
        pltpu.async_copy(src_hbm.at[pl.ds(e0 + i * _C, _C)], idx, sem)

    def ldwait(idx, sem):
        pltpu.make_async_copy(src_hbm.at[pl.ds(e0, _C)], idx, sem).wait()

    def fire(idx, rows, sem):
        pltpu.async_copy(g_hbm.at[idx], rows, sem)

    def drain(idx, rows, sem):
        pltpu.make_async_copy(g_hbm.at[idx], rows, sem).wait()

    ld(0, src_a, sem_ia)
    ldwait(src_a, sem_ia)
    fire(src_a, rows_a, sem_ga)
    ld(1, src_b, sem_ib)

    def body(k, carry):
        i = 2 * k
        ldwait(src_b, sem_ib)
        fire(src_b, rows_b, sem_gb)
        drain(src_a, rows_a, sem_ga)
        pltpu.sync_copy(rows_a, acc_sh.at[dst2d.at[i]], add=True)

        @pl.when(i + 2 < _RW)
        def _():
            ld(i + 2, src_a, sem_ia)

        drain(src_b, rows_b, sem_gb)
        pltpu.sync_copy(rows_b, acc_sh.at[dst2d.at[i + 1]], add=True)

        @pl.when(i + 2 < _RW)
        def _():
            ldwait(src_a, sem_ia)
            fire(src_a, rows_a, sem_ga)

        @pl.when(i + 3 < _RW)
        def _():
            ld(i + 3, src_b, sem_ib)

        return carry

    lax.fori_loop(0, _RW // 2, body, 0)
    plsc.subcore_barrier()

    for k in range(4):
        w0 = s * _RPT + k * _C
        pltpu.sync_copy(acc_sh.at[pl.ds(w0, _C)], rows_a)
        pltpu.sync_copy(rows_a, out_hbm.at[c, pl.ds(w0, _C)])
    w0 = s * _RPT + 4 * _C
    pltpu.sync_copy(acc_sh.at[pl.ds(w0, _RPT - 4 * _C)],
                    rows_a.at[pl.ds(0, _RPT - 4 * _C)])
    pltpu.sync_copy(rows_a.at[pl.ds(0, _RPT - 4 * _C)],
                    out_hbm.at[c, pl.ds(w0, _RPT - 4 * _C)])

    @pl.when(s == _NS - 1)
    def _():
        pltpu.sync_copy(acc_sh.at[pl.ds(_NS * _RPT, _RTAIL)],
                        rows_b.at[pl.ds(0, _RTAIL)])
        pltpu.sync_copy(rows_b.at[pl.ds(0, _RTAIL)],
                        out_hbm.at[c, pl.ds(_NS * _RPT, _RTAIL)])


_BR = 2000


def _norm_mul_body(degp_ref, feat_ref, norm_ref, g1_ref):
    d = degp_ref[0] + degp_ref[1]
    nv = lax.rsqrt(jnp.maximum(d, 1.0))
    norm_ref[...] = nv
    g1_ref[...] = feat_ref[...] * nv


def _mid_body(p_ref, norm_ref, g2_ref):
    nv = norm_ref[...]
    g2_ref[...] = (p_ref[0] + p_ref[1]) * (nv * nv)


def _fc_body(p_ref, norm_ref, w_ref, b_ref, out_ref):
    h = (p_ref[0] + p_ref[1]) * norm_ref[...]
    out_ref[...] = (
        jnp.dot(h, w_ref[...], preferred_element_type=_f32) + b_ref[...]
    )


_norm_call = pl.pallas_call(
    _norm_mul_body,
    grid=(_N // _BR,),
    in_specs=[
        pl.BlockSpec((_NC, _BR, 1), lambda i: (0, i, 0)),
        pl.BlockSpec((_BR, _D), lambda i: (i, 0)),
    ],
    out_specs=[
        pl.BlockSpec((_BR, 1), lambda i: (i, 0)),
        pl.BlockSpec((_BR, _D), lambda i: (i, 0)),
    ],
    out_shape=[
        jax.ShapeDtypeStruct((_N, 1), _f32),
        jax.ShapeDtypeStruct((_N, _D), _f32),
    ],
)

_mid_call = pl.pallas_call(
    _mid_body,
    grid=(_N // _BR,),
    in_specs=[
        pl.BlockSpec((_NC, _BR, _D), lambda i: (0, i, 0)),
        pl.BlockSpec((_BR, 1), lambda i: (i, 0)),
    ],
    out_specs=pl.BlockSpec((_BR, _D), lambda i: (i, 0)),
    out_shape=jax.ShapeDtypeStruct((_N, _D), _f32),
)

_fc_call = pl.pallas_call(
    _fc_body,
    grid=(_N // _BR,),
    in_specs=[
        pl.BlockSpec((_NC, _BR, _D), lambda i: (0, i, 0)),
        pl.BlockSpec((_BR, 1), lambda i: (i, 0)),
        pl.BlockSpec((_D, _D), lambda i: (0, 0)),
        pl.BlockSpec((1, _D), lambda i: (0, 0)),
    ],
    out_specs=pl.BlockSpec((_BR, _D), lambda i: (i, 0)),
    out_shape=jax.ShapeDtypeStruct((_N, _D), _f32),
)


def kernel(feat, edge_index, W, b):
    pad_src = jnp.zeros((_EPAD,), jnp.int32)
    pad_dst = jnp.full((_EPAD,), _N, jnp.int32)
    src = jnp.concatenate([edge_index[0], pad_src])
    dst = jnp.concatenate([edge_index[1], pad_dst]).reshape(_ER, _C)
    degp = _deg_kernel(dst)
    norm, g1 = _norm_call(degp.reshape(_NC, _N, 1), feat)
    p1 = _hop_kernel(g1, src, dst)
    g2 = _mid_call(p1, norm)
    p2 = _hop_kernel(g2, src, dst)
    out = _fc_call(p2, norm, W, b.reshape(1, _D))
    return out


# speedup vs baseline: 1.0769x; 1.0769x over previous
import functools

import jax
import jax.numpy as jnp
from jax import lax
from jax.experimental import pallas as pl
from jax.experimental.pallas import tpu as pltpu
from jax.experimental.pallas import tpu_sc as plsc

_N = 10000
_E = 320000
_D = 128
_NC = 2
_NS = 16
_NW = _NC * _NS
_C = 128
_ER = 2560
_EPAD = _ER * _C - _E
_RW = _ER // _NW
_PR = 256
_RC = 104
_RPT = 624
_RTAIL = _N - _NS * _RPT
_DPT = 624
_DTAIL = _N - _NS * _DPT

_mesh = plsc.VectorSubcoreMesh(core_axis_name="c", subcore_axis_name="s")
_f32 = jnp.float32


@functools.partial(
    pl.kernel,
    out_type=jax.ShapeDtypeStruct((_NC * _N,), _f32),
    mesh=_mesh,
    scratch_types=[
        pltpu.VMEM((_RW, _C), jnp.int32),
        pltpu.VMEM((_C,), _f32),
        pltpu.VMEM((_DPT,), _f32),
        pltpu.VMEM_SHARED((_N + _PR,), _f32),
        pltpu.SemaphoreType.DMA,
        pltpu.SemaphoreType.DMA,
    ],
)
def _deg_kernel(dst_hbm, out_hbm, dst2d, ones_v, buf_v, acc_sh,
                sem_i, sem_s):
    c = lax.axis_index("c")
    s = lax.axis_index("s")
    wid = c * _NS + s
    pltpu.async_copy(dst_hbm.at[pl.ds(wid * _RW, _RW)], dst2d, sem_i)

    for j in range(_C // 16):
        ones_v[pl.ds(j * 16, 16)] = jnp.ones((16,), _f32)
    for j in range(_DPT // 16):
        buf_v[pl.ds(j * 16, 16)] = jnp.zeros((16,), _f32)
    pltpu.sync_copy(buf_v, acc_sh.at[pl.ds(s * _DPT, _DPT)])

    @pl.when(s == _NS - 1)
    def _():
        pltpu.sync_copy(buf_v.at[pl.ds(0, _DTAIL)],
                        acc_sh.at[pl.ds(_NS * _DPT, _DTAIL)])

    pltpu.make_async_copy(dst_hbm.at[pl.ds(wid * _RW, _RW)],
                          dst2d, sem_i).wait()
    plsc.subcore_barrier()

    def body(i, carry):
        pltpu.async_copy(ones_v, acc_sh.at[dst2d.at[i]], sem_s, add=True)
        return carry

    lax.fori_loop(0, _RW, body, 0)

    def drain(i, carry):
        pltpu.make_async_copy(ones_v, acc_sh.at[dst2d.at[0]], sem_s).wait()
        return carry

    lax.fori_loop(0, _RW, drain, 0)
    plsc.subcore_barrier()

    pltpu.sync_copy(acc_sh.at[pl.ds(s * _DPT, _DPT)], buf_v)
    pltpu.sync_copy(buf_v, out_hbm.at[pl.ds(c * _N + s * _DPT, _DPT)])

    @pl.when(s == _NS - 1)
    def _():
        pltpu.sync_copy(acc_sh.at[pl.ds(_NS * _DPT, _DTAIL)],
                        buf_v.at[pl.ds(0, _DTAIL)])
        pltpu.sync_copy(buf_v.at[pl.ds(0, _DTAIL)],
                        out_hbm.at[pl.ds(c * _N + _NS * _DPT, _DTAIL)])


@functools.partial(
    pl.kernel,
    out_type=jax.ShapeDtypeStruct((_NC, _N, _D), _f32),
    mesh=_mesh,
    scratch_types=[
        pltpu.VMEM((_C,), jnp.int32),
        pltpu.VMEM((_C,), jnp.int32),
        pltpu.VMEM((_RW, _C), jnp.int32),
        pltpu.VMEM((_C, _D), _f32),
        pltpu.VMEM((_C, _D), _f32),
        pltpu.VMEM_SHARED((_N + _PR, _D), _f32),
        pltpu.SemaphoreType.DMA,
        pltpu.SemaphoreType.DMA,
        pltpu.SemaphoreType.DMA,
        pltpu.SemaphoreType.DMA,
        pltpu.SemaphoreType.DMA,
    ],
)
def _hop_kernel(g_hbm, src_hbm, dst_hbm, out_hbm,
                src_a, src_b, dst2d, rows_a, rows_b, acc_sh,
                sem_d, sem_ia, sem_ib, sem_ga, sem_gb):
    c = lax.axis_index("c")
    s = lax.axis_index("s")
    wid = c * _NS + s
    e0 = wid * _RW * _C
    pltpu.async_copy(dst_hbm.at[pl.ds(wid * _RW, _RW)], dst2d, sem_d)

    def zrow(r, carry):
        for j in range(_D // 16):
            rows_a[r, pl.ds(j * 16, 16)] = jnp.zeros((16,), _f32)
        return carry

    lax.fori_loop(0, _C, zrow, 0)
    r0 = s * _RPT
    for k in range(4):
        pltpu.sync_copy(rows_a, acc_sh.at[pl.ds(r0 + k * _C, _C)])
    pltpu.sync_copy(rows_a.at[pl.ds(0, _RPT - 4 * _C)],
                    acc_sh.at[pl.ds(r0 + 4 * _C, _RPT - 4 * _C)])

    @pl.when(s == _NS - 1)
    def _():
        pltpu.sync_copy(rows_a.at[pl.ds(0, _RTAIL)],
                        acc_sh.at[pl.ds(_NS * _RPT, _RTAIL)])

    pltpu.make_async_copy(dst_hbm.at[pl.ds(wid * _RW, _RW)],
                          dst2d, sem_d).wait()
    plsc.subcore_barrier()

    def ld(i, idx, sem):
        pltpu.async_copy(src_hbm.at[pl.ds(e0 + i * _C, _C)], idx, sem)

    def ldwait(idx, sem):
        pltpu.make_async_copy(src_hbm.at[pl.ds(e0, _C)], idx, sem).wait()

    def fire(idx, rows, sem):
        pltpu.async_copy(g_hbm.at[idx], rows, sem)

    def drain(idx, rows, sem):
        pltpu.make_async_copy(g_hbm.at[idx], rows, sem).wait()

    ld(0, src_a, sem_ia)
    ldwait(src_a, sem_ia)
    fire(src_a, rows_a, sem_ga)
    ld(1, src_b, sem_ib)

    def body(k, carry):
        i = 2 * k
        ldwait(src_b, sem_ib)
        fire(src_b, rows_b, sem_gb)
        drain(src_a, rows_a, sem_ga)
        pltpu.sync_copy(rows_a, acc_sh.at[dst2d.at[i]], add=True)

        @pl.when(i + 2 < _RW)
        def _():
            ld(i + 2, src_a, sem_ia)

        drain(src_b, rows_b, sem_gb)
        pltpu.sync_copy(rows_b, acc_sh.at[dst2d.at[i + 1]], add=True)

        @pl.when(i + 2 < _RW)
        def _():
            ldwait(src_a, sem_ia)
            fire(src_a, rows_a, sem_ga)

        @pl.when(i + 3 < _RW)
        def _():
            ld(i + 3, src_b, sem_ib)

        return carry

    lax.fori_loop(0, _RW // 2, body, 0)
    plsc.subcore_barrier()

    for k in range(4):
        w0 = s * _RPT + k * _C
        pltpu.sync_copy(acc_sh.at[pl.ds(w0, _C)], rows_a)
        pltpu.sync_copy(rows_a, out_hbm.at[c, pl.ds(w0, _C)])
    w0 = s * _RPT + 4 * _C
    pltpu.sync_copy(acc_sh.at[pl.ds(w0, _RPT - 4 * _C)],
                    rows_a.at[pl.ds(0, _RPT - 4 * _C)])
    pltpu.sync_copy(rows_a.at[pl.ds(0, _RPT - 4 * _C)],
                    out_hbm.at[c, pl.ds(w0, _RPT - 4 * _C)])

    @pl.when(s == _NS - 1)
    def _():
        pltpu.sync_copy(acc_sh.at[pl.ds(_NS * _RPT, _RTAIL)],
                        rows_b.at[pl.ds(0, _RTAIL)])
        pltpu.sync_copy(rows_b.at[pl.ds(0, _RTAIL)],
                        out_hbm.at[c, pl.ds(_NS * _RPT, _RTAIL)])


_BR = 2000


def _norm_mul_body(degp_ref, feat_ref, norm_ref, g1_ref):
    d = degp_ref[0] + degp_ref[1]
    nv = lax.rsqrt(jnp.maximum(d, 1.0))
    norm_ref[...] = nv
    g1_ref[...] = feat_ref[...] * nv


def _mid_body(p_ref, norm_ref, g2_ref):
    nv = norm_ref[...]
    g2_ref[...] = (p_ref[0] + p_ref[1]) * (nv * nv)


def _fc_body(p_ref, norm_ref, w_ref, b_ref, out_ref):
    h = (p_ref[0] + p_ref[1]) * norm_ref[...]
    out_ref[...] = (
        jnp.dot(h, w_ref[...], preferred_element_type=_f32) + b_ref[...]
    )


_norm_call = pl.pallas_call(
    _norm_mul_body,
    grid=(_N // _BR,),
    in_specs=[
        pl.BlockSpec((_NC, _BR, 1), lambda i: (0, i, 0)),
        pl.BlockSpec((_BR, _D), lambda i: (i, 0)),
    ],
    out_specs=[
        pl.BlockSpec((_BR, 1), lambda i: (i, 0)),
        pl.BlockSpec((_BR, _D), lambda i: (i, 0)),
    ],
    out_shape=[
        jax.ShapeDtypeStruct((_N, 1), _f32),
        jax.ShapeDtypeStruct((_N, _D), _f32),
    ],
)

_mid_call = pl.pallas_call(
    _mid_body,
    grid=(_N // _BR,),
    in_specs=[
        pl.BlockSpec((_NC, _BR, _D), lambda i: (0, i, 0)),
        pl.BlockSpec((_BR, 1), lambda i: (i, 0)),
    ],
    out_specs=pl.BlockSpec((_BR, _D), lambda i: (i, 0)),
    out_shape=jax.ShapeDtypeStruct((_N, _D), _f32),
)

_fc_call = pl.pallas_call(
    _fc_body,
    grid=(_N // _BR,),
    in_specs=[
        pl.BlockSpec((_NC, _BR, _D), lambda i: (0, i, 0)),
        pl.BlockSpec((_BR, 1), lambda i: (i, 0)),
        pl.BlockSpec((_D, _D), lambda i: (0, 0)),
        pl.BlockSpec((1, _D), lambda i: (0, 0)),
    ],
    out_specs=pl.BlockSpec((_BR, _D), lambda i: (i, 0)),
    out_shape=jax.ShapeDtypeStruct((_N, _D), _f32),
)


def kernel(feat, edge_index, W, b):
    pad_src = jnp.zeros((_EPAD,), jnp.int32)
    pad_dst = _N + (jnp.arange(_EPAD, dtype=jnp.int32) % _PR)
    src = jnp.concatenate([edge_index[0], pad_src])
    dst = jnp.concatenate([edge_index[1], pad_dst]).reshape(_ER, _C)
    degp = _deg_kernel(dst)
    norm, g1 = _norm_call(degp.reshape(_NC, _N, 1), feat)
    p1 = _hop_kernel(g1, src, dst)
    g2 = _mid_call(p1, norm)
    p2 = _hop_kernel(g2, src, dst)
    out = _fc_call(p2, norm, W, b.reshape(1, _D))
    return out


# speedup vs baseline: 3.2030x; 2.9744x over previous
import functools

import jax
import jax.numpy as jnp
from jax import lax
from jax.experimental import pallas as pl
from jax.experimental.pallas import tpu as pltpu
from jax.experimental.pallas import tpu_sc as plsc

_N = 10000
_E = 320000
_D = 128
_NC = 2
_NS = 16
_NW = _NC * _NS
_C = 128
_ER = 2560
_EPAD = _ER * _C - _E
_RW = _ER // _NW
_PR = 256
_RC = 104
_RPT = 624
_RTAIL = _N - _NS * _RPT
_DPT = 624
_DTAIL = _N - _NS * _DPT

_mesh = plsc.VectorSubcoreMesh(core_axis_name="c", subcore_axis_name="s")
_f32 = jnp.float32


@functools.partial(
    pl.kernel,
    out_type=jax.ShapeDtypeStruct((_NC * _N,), _f32),
    mesh=_mesh,
    scratch_types=[
        pltpu.VMEM((_RW, _C), jnp.int32),
        pltpu.VMEM((_C,), _f32),
        pltpu.VMEM((_DPT,), _f32),
        pltpu.VMEM_SHARED((_N + _PR,), _f32),
        pltpu.SemaphoreType.DMA,
        pltpu.SemaphoreType.DMA,
    ],
)
def _deg_kernel(dst_hbm, out_hbm, dst2d, ones_v, buf_v, acc_sh,
                sem_i, sem_s):
    c = lax.axis_index("c")
    s = lax.axis_index("s")
    wid = c * _NS + s
    pltpu.async_copy(dst_hbm.at[pl.ds(wid * _RW, _RW)], dst2d, sem_i)

    for j in range(_C // 16):
        ones_v[pl.ds(j * 16, 16)] = jnp.ones((16,), _f32)
    for j in range(_DPT // 16):
        buf_v[pl.ds(j * 16, 16)] = jnp.zeros((16,), _f32)
    pltpu.sync_copy(buf_v, acc_sh.at[pl.ds(s * _DPT, _DPT)])

    @pl.when(s == _NS - 1)
    def _():
        pltpu.sync_copy(buf_v.at[pl.ds(0, _DTAIL)],
                        acc_sh.at[pl.ds(_NS * _DPT, _DTAIL)])

    pltpu.make_async_copy(dst_hbm.at[pl.ds(wid * _RW, _RW)],
                          dst2d, sem_i).wait()
    plsc.subcore_barrier()

    def body(i, carry):
        pltpu.async_copy(ones_v, acc_sh.at[dst2d.at[i]], sem_s, add=True)
        return carry

    lax.fori_loop(0, _RW, body, 0)

    def drain(i, carry):
        pltpu.make_async_copy(ones_v, acc_sh.at[dst2d.at[0]], sem_s).wait()
        return carry

    lax.fori_loop(0, _RW, drain, 0)
    plsc.subcore_barrier()

    pltpu.sync_copy(acc_sh.at[pl.ds(s * _DPT, _DPT)], buf_v)
    pltpu.sync_copy(buf_v, out_hbm.at[pl.ds(c * _N + s * _DPT, _DPT)])

    @pl.when(s == _NS - 1)
    def _():
        pltpu.sync_copy(acc_sh.at[pl.ds(_NS * _DPT, _DTAIL)],
                        buf_v.at[pl.ds(0, _DTAIL)])
        pltpu.sync_copy(buf_v.at[pl.ds(0, _DTAIL)],
                        out_hbm.at[pl.ds(c * _N + _NS * _DPT, _DTAIL)])


@functools.partial(
    pl.kernel,
    out_type=jax.ShapeDtypeStruct((_NC, _N, _D), _f32),
    mesh=_mesh,
    scratch_types=[
        pltpu.VMEM((_C,), jnp.int32),
        pltpu.VMEM((_C,), jnp.int32),
        pltpu.VMEM((_RW, _C), jnp.int32),
        pltpu.VMEM((_C, _D), _f32),
        pltpu.VMEM((_C, _D), _f32),
        pltpu.VMEM_SHARED((_N + _PR, _D), _f32),
        pltpu.SemaphoreType.DMA,
        pltpu.SemaphoreType.DMA,
        pltpu.SemaphoreType.DMA,
        pltpu.SemaphoreType.DMA,
        pltpu.SemaphoreType.DMA,
    ],
)
def _hop_kernel(g_hbm, src_hbm, dst_hbm, out_hbm,
                src_a, src_b, dst2d, rows_a, rows_b, acc_sh,
                sem_d, sem_ia, sem_ib, sem_ga, sem_gb):
    c = lax.axis_index("c")
    s = lax.axis_index("s")
    wid = c * _NS + s
    e0 = wid * _RW * _C
    pltpu.async_copy(dst_hbm.at[pl.ds(wid * _RW, _RW)], dst2d, sem_d)

    def zrow(r, carry):
        for j in range(_D // 16):
            rows_a[r, pl.ds(j * 16, 16)] = jnp.zeros((16,), _f32)
        return carry

    lax.fori_loop(0, _C, zrow, 0)
    r0 = s * _RPT
    for k in range(4):
        pltpu.sync_copy(rows_a, acc_sh.at[pl.ds(r0 + k * _C, _C)])
    pltpu.sync_copy(rows_a.at[pl.ds(0, _RPT - 4 * _C)],
                    acc_sh.at[pl.ds(r0 + 4 * _C, _RPT - 4 * _C)])

    @pl.when(s == _NS - 1)
    def _():
        pltpu.sync_copy(rows_a.at[pl.ds(0, _RTAIL)],
                        acc_sh.at[pl.ds(_NS * _RPT, _RTAIL)])

    pltpu.make_async_copy(dst_hbm.at[pl.ds(wid * _RW, _RW)],
                          dst2d, sem_d).wait()
    plsc.subcore_barrier()

    def ld(i, idx, sem):
        pltpu.async_copy(src_hbm.at[pl.ds(e0 + i * _C, _C)], idx, sem)

    def ldwait(idx, sem):
        pltpu.make_async_copy(src_hbm.at[pl.ds(e0, _C)], idx, sem).wait()

    def fire(idx, rows, sem):
        pltpu.async_copy(g_hbm.at[idx], rows, sem)

    def drain(idx, rows, sem):
        pltpu.make_async_copy(g_hbm.at[idx], rows, sem).wait()

    ld(0, src_a, sem_ia)
    ldwait(src_a, sem_ia)
    fire(src_a, rows_a, sem_ga)
    ld(1, src_b, sem_ib)

    def body(k, carry):
        i = 2 * k
        ldwait(src_b, sem_ib)
        fire(src_b, rows_b, sem_gb)
        drain(src_a, rows_a, sem_ga)
        pltpu.sync_copy(rows_a, acc_sh.at[dst2d.at[i]], add=True)

        @pl.when(i + 2 < _RW)
        def _():
            ld(i + 2, src_a, sem_ia)

        drain(src_b, rows_b, sem_gb)
        pltpu.sync_copy(rows_b, acc_sh.at[dst2d.at[i + 1]], add=True)

        @pl.when(i + 2 < _RW)
        def _():
            ldwait(src_a, sem_ia)
            fire(src_a, rows_a, sem_ga)

        @pl.when(i + 3 < _RW)
        def _():
            ld(i + 3, src_b, sem_ib)

        return carry

    lax.fori_loop(0, _RW // 2, body, 0)
    plsc.subcore_barrier()

    for k in range(4):
        w0 = s * _RPT + k * _C
        pltpu.sync_copy(acc_sh.at[pl.ds(w0, _C)], rows_a)
        pltpu.sync_copy(rows_a, out_hbm.at[c, pl.ds(w0, _C)])
    w0 = s * _RPT + 4 * _C
    pltpu.sync_copy(acc_sh.at[pl.ds(w0, _RPT - 4 * _C)],
                    rows_a.at[pl.ds(0, _RPT - 4 * _C)])
    pltpu.sync_copy(rows_a.at[pl.ds(0, _RPT - 4 * _C)],
                    out_hbm.at[c, pl.ds(w0, _RPT - 4 * _C)])

    @pl.when(s == _NS - 1)
    def _():
        pltpu.sync_copy(acc_sh.at[pl.ds(_NS * _RPT, _RTAIL)],
                        rows_b.at[pl.ds(0, _RTAIL)])
        pltpu.sync_copy(rows_b.at[pl.ds(0, _RTAIL)],
                        out_hbm.at[c, pl.ds(_NS * _RPT, _RTAIL)])


_BR = 2000


def _norm_mul_body(degp_ref, feat_ref, norm_ref, g1_ref):
    d = degp_ref[0] + degp_ref[1]
    nv = lax.rsqrt(jnp.maximum(d, 1.0))
    norm_ref[...] = nv
    g1_ref[...] = feat_ref[...] * nv


def _mid_body(p_ref, norm_ref, g2_ref):
    nv = norm_ref[...]
    g2_ref[...] = (p_ref[0] + p_ref[1]) * (nv * nv)


def _fc_body(p_ref, norm_ref, w_ref, b_ref, out_ref):
    h = (p_ref[0] + p_ref[1]) * norm_ref[...]
    out_ref[...] = (
        jnp.dot(h, w_ref[...], preferred_element_type=_f32) + b_ref[...]
    )


_norm_call = pl.pallas_call(
    _norm_mul_body,
    grid=(_N // _BR,),
    in_specs=[
        pl.BlockSpec((_NC, _BR, 1), lambda i: (0, i, 0)),
        pl.BlockSpec((_BR, _D), lambda i: (i, 0)),
    ],
    out_specs=[
        pl.BlockSpec((_BR, 1), lambda i: (i, 0)),
        pl.BlockSpec((_BR, _D), lambda i: (i, 0)),
    ],
    out_shape=[
        jax.ShapeDtypeStruct((_N, 1), _f32),
        jax.ShapeDtypeStruct((_N, _D), _f32),
    ],
)

_mid_call = pl.pallas_call(
    _mid_body,
    grid=(_N // _BR,),
    in_specs=[
        pl.BlockSpec((_NC, _BR, _D), lambda i: (0, i, 0)),
        pl.BlockSpec((_BR, 1), lambda i: (i, 0)),
    ],
    out_specs=pl.BlockSpec((_BR, _D), lambda i: (i, 0)),
    out_shape=jax.ShapeDtypeStruct((_N, _D), _f32),
)

_fc_call = pl.pallas_call(
    _fc_body,
    grid=(_N // _BR,),
    in_specs=[
        pl.BlockSpec((_NC, _BR, _D), lambda i: (0, i, 0)),
        pl.BlockSpec((_BR, 1), lambda i: (i, 0)),
        pl.BlockSpec((_D, _D), lambda i: (0, 0)),
        pl.BlockSpec((1, _D), lambda i: (0, 0)),
    ],
    out_specs=pl.BlockSpec((_BR, _D), lambda i: (i, 0)),
    out_shape=jax.ShapeDtypeStruct((_N, _D), _f32),
)


def kernel(feat, edge_index, W, b):
    pad_iota = jnp.arange(_EPAD, dtype=jnp.int32)
    pad_src = pad_iota % _N
    pad_dst = _N + (pad_iota % _PR)
    src = jnp.concatenate([edge_index[0], pad_src])
    dst = jnp.concatenate([edge_index[1], pad_dst]).reshape(_ER, _C)
    degp = _deg_kernel(dst)
    norm, g1 = _norm_call(degp.reshape(_NC, _N, 1), feat)
    p1 = _hop_kernel(g1, src, dst)
    g2 = _mid_call(p1, norm)
    p2 = _hop_kernel(g2, src, dst)
    out = _fc_call(p2, norm, W, b.reshape(1, _D))
    return out


# speedup vs baseline: 3.5429x; 1.1061x over previous
import functools

import jax
import jax.numpy as jnp
from jax import lax
from jax.experimental import pallas as pl
from jax.experimental.pallas import tpu as pltpu
from jax.experimental.pallas import tpu_sc as plsc

_N = 10000
_E = 320000
_D = 128
_NC = 2
_NS = 16
_NW = _NC * _NS
_C = 128
_ER = 2560
_EPAD = _ER * _C - _E
_RW = _ER // _NW
_PR = 256
_RC = 104
_RPT = 624
_RTAIL = _N - _NS * _RPT
_DPT = 624
_DTAIL = _N - _NS * _DPT

_mesh = plsc.VectorSubcoreMesh(core_axis_name="c", subcore_axis_name="s")
_f32 = jnp.float32


@functools.partial(
    pl.kernel,
    out_type=jax.ShapeDtypeStruct((_NC * _N,), _f32),
    mesh=_mesh,
    scratch_types=[
        pltpu.VMEM((_RW, _C), jnp.int32),
        pltpu.VMEM((_C,), _f32),
        pltpu.VMEM((_DPT,), _f32),
        pltpu.VMEM_SHARED((_N + _PR,), _f32),
        pltpu.SemaphoreType.DMA,
        pltpu.SemaphoreType.DMA,
    ],
)
def _deg_kernel(dst_hbm, out_hbm, dst2d, ones_v, buf_v, acc_sh,
                sem_i, sem_s):
    c = lax.axis_index("c")
    s = lax.axis_index("s")
    wid = c * _NS + s
    pltpu.async_copy(dst_hbm.at[pl.ds(wid * _RW, _RW)], dst2d, sem_i)

    for j in range(_C // 16):
        ones_v[pl.ds(j * 16, 16)] = jnp.ones((16,), _f32)
    for j in range(_DPT // 16):
        buf_v[pl.ds(j * 16, 16)] = jnp.zeros((16,), _f32)
    pltpu.sync_copy(buf_v, acc_sh.at[pl.ds(s * _DPT, _DPT)])

    @pl.when(s == _NS - 1)
    def _():
        pltpu.sync_copy(buf_v.at[pl.ds(0, _DTAIL)],
                        acc_sh.at[pl.ds(_NS * _DPT, _DTAIL)])

    pltpu.make_async_copy(dst_hbm.at[pl.ds(wid * _RW, _RW)],
                          dst2d, sem_i).wait()
    plsc.subcore_barrier()

    def body(i, carry):
        pltpu.async_copy(ones_v, acc_sh.at[dst2d.at[i]], sem_s, add=True)
        return carry

    lax.fori_loop(0, _RW, body, 0)

    def drain(i, carry):
        pltpu.make_async_copy(ones_v, acc_sh.at[dst2d.at[0]], sem_s).wait()
        return carry

    lax.fori_loop(0, _RW, drain, 0)
    plsc.subcore_barrier()

    pltpu.sync_copy(acc_sh.at[pl.ds(s * _DPT, _DPT)], buf_v)
    pltpu.sync_copy(buf_v, out_hbm.at[pl.ds(c * _N + s * _DPT, _DPT)])

    @pl.when(s == _NS - 1)
    def _():
        pltpu.sync_copy(acc_sh.at[pl.ds(_NS * _DPT, _DTAIL)],
                        buf_v.at[pl.ds(0, _DTAIL)])
        pltpu.sync_copy(buf_v.at[pl.ds(0, _DTAIL)],
                        out_hbm.at[pl.ds(c * _N + _NS * _DPT, _DTAIL)])


@functools.partial(
    pl.kernel,
    out_type=jax.ShapeDtypeStruct((_NC, _N, _D), _f32),
    mesh=_mesh,
    scratch_types=[
        pltpu.VMEM((_C,), jnp.int32),
        pltpu.VMEM((_C,), jnp.int32),
        pltpu.VMEM((_RW, _C), jnp.int32),
        pltpu.VMEM((_C, _D), _f32),
        pltpu.VMEM((_C, _D), _f32),
        pltpu.VMEM_SHARED((_N + _PR, _D), _f32),
        pltpu.SemaphoreType.DMA,
        pltpu.SemaphoreType.DMA,
        pltpu.SemaphoreType.DMA,
        pltpu.SemaphoreType.DMA,
        pltpu.SemaphoreType.DMA,
        pltpu.SemaphoreType.DMA,
        pltpu.SemaphoreType.DMA,
    ],
)
def _hop_kernel(g_hbm, src_hbm, dst_hbm, out_hbm,
                src_a, src_b, dst2d, rows_a, rows_b, acc_sh,
                sem_d, sem_ia, sem_ib, sem_ga, sem_gb, sem_sa, sem_sb):
    c = lax.axis_index("c")
    s = lax.axis_index("s")
    wid = c * _NS + s
    e0 = wid * _RW * _C
    pltpu.async_copy(dst_hbm.at[pl.ds(wid * _RW, _RW)], dst2d, sem_d)

    def zrow(r, carry):
        for j in range(_D // 16):
            rows_a[r, pl.ds(j * 16, 16)] = jnp.zeros((16,), _f32)
        return carry

    lax.fori_loop(0, _C, zrow, 0)
    r0 = s * _RPT
    for k in range(4):
        pltpu.sync_copy(rows_a, acc_sh.at[pl.ds(r0 + k * _C, _C)])
    pltpu.sync_copy(rows_a.at[pl.ds(0, _RPT - 4 * _C)],
                    acc_sh.at[pl.ds(r0 + 4 * _C, _RPT - 4 * _C)])

    @pl.when(s == _NS - 1)
    def _():
        pltpu.sync_copy(rows_a.at[pl.ds(0, _RTAIL)],
                        acc_sh.at[pl.ds(_NS * _RPT, _RTAIL)])

    pltpu.make_async_copy(dst_hbm.at[pl.ds(wid * _RW, _RW)],
                          dst2d, sem_d).wait()
    plsc.subcore_barrier()

    def ld(i, idx, sem):
        pltpu.async_copy(src_hbm.at[pl.ds(e0 + i * _C, _C)], idx, sem)

    def ldwait(idx, sem):
        pltpu.make_async_copy(src_hbm.at[pl.ds(e0, _C)], idx, sem).wait()

    def fire(idx, rows, sem):
        pltpu.async_copy(g_hbm.at[idx], rows, sem)

    def drain(idx, rows, sem):
        pltpu.make_async_copy(g_hbm.at[idx], rows, sem).wait()

    ld(0, src_a, sem_ia)
    ldwait(src_a, sem_ia)
    fire(src_a, rows_a, sem_ga)
    ld(1, src_b, sem_ib)

    def scat_start(rows, i, sem):
        pltpu.async_copy(rows, acc_sh.at[dst2d.at[i]], sem, add=True)

    def scat_wait(rows, sem):
        pltpu.make_async_copy(rows, acc_sh.at[dst2d.at[0]], sem).wait()

    def body(k, carry):
        i = 2 * k
        ldwait(src_b, sem_ib)
        fire(src_b, rows_b, sem_gb)
        drain(src_a, rows_a, sem_ga)
        scat_start(rows_a, i, sem_sa)

        @pl.when(i + 2 < _RW)
        def _():
            ld(i + 2, src_a, sem_ia)

        drain(src_b, rows_b, sem_gb)
        scat_start(rows_b, i + 1, sem_sb)

        @pl.when(i + 2 < _RW)
        def _():
            scat_wait(rows_a, sem_sa)
            ldwait(src_a, sem_ia)
            fire(src_a, rows_a, sem_ga)

        @pl.when(i + 3 < _RW)
        def _():
            ld(i + 3, src_b, sem_ib)

        scat_wait(rows_b, sem_sb)
        return carry

    lax.fori_loop(0, _RW // 2, body, 0)
    scat_wait(rows_a, sem_sa)
    plsc.subcore_barrier()

    for k in range(4):
        w0 = s * _RPT + k * _C
        pltpu.sync_copy(acc_sh.at[pl.ds(w0, _C)], rows_a)
        pltpu.sync_copy(rows_a, out_hbm.at[c, pl.ds(w0, _C)])
    w0 = s * _RPT + 4 * _C
    pltpu.sync_copy(acc_sh.at[pl.ds(w0, _RPT - 4 * _C)],
                    rows_a.at[pl.ds(0, _RPT - 4 * _C)])
    pltpu.sync_copy(rows_a.at[pl.ds(0, _RPT - 4 * _C)],
                    out_hbm.at[c, pl.ds(w0, _RPT - 4 * _C)])

    @pl.when(s == _NS - 1)
    def _():
        pltpu.sync_copy(acc_sh.at[pl.ds(_NS * _RPT, _RTAIL)],
                        rows_b.at[pl.ds(0, _RTAIL)])
        pltpu.sync_copy(rows_b.at[pl.ds(0, _RTAIL)],
                        out_hbm.at[c, pl.ds(_NS * _RPT, _RTAIL)])


_BR = 2000


def _norm_mul_body(degp_ref, feat_ref, norm_ref, g1_ref):
    d = degp_ref[0] + degp_ref[1]
    nv = lax.rsqrt(jnp.maximum(d, 1.0))
    norm_ref[...] = nv
    g1_ref[...] = feat_ref[...] * nv


def _mid_body(p_ref, norm_ref, g2_ref):
    nv = norm_ref[...]
    g2_ref[...] = (p_ref[0] + p_ref[1]) * (nv * nv)


def _fc_body(p_ref, norm_ref, w_ref, b_ref, out_ref):
    h = (p_ref[0] + p_ref[1]) * norm_ref[...]
    out_ref[...] = (
        jnp.dot(h, w_ref[...], preferred_element_type=_f32) + b_ref[...]
    )


_norm_call = pl.pallas_call(
    _norm_mul_body,
    grid=(_N // _BR,),
    in_specs=[
        pl.BlockSpec((_NC, _BR, 1), lambda i: (0, i, 0)),
        pl.BlockSpec((_BR, _D), lambda i: (i, 0)),
    ],
    out_specs=[
        pl.BlockSpec((_BR, 1), lambda i: (i, 0)),
        pl.BlockSpec((_BR, _D), lambda i: (i, 0)),
    ],
    out_shape=[
        jax.ShapeDtypeStruct((_N, 1), _f32),
        jax.ShapeDtypeStruct((_N, _D), _f32),
    ],
)

_mid_call = pl.pallas_call(
    _mid_body,
    grid=(_N // _BR,),
    in_specs=[
        pl.BlockSpec((_NC, _BR, _D), lambda i: (0, i, 0)),
        pl.BlockSpec((_BR, 1), lambda i: (i, 0)),
    ],
    out_specs=pl.BlockSpec((_BR, _D), lambda i: (i, 0)),
    out_shape=jax.ShapeDtypeStruct((_N, _D), _f32),
)

_fc_call = pl.pallas_call(
    _fc_body,
    grid=(_N // _BR,),
    in_specs=[
        pl.BlockSpec((_NC, _BR, _D), lambda i: (0, i, 0)),
        pl.BlockSpec((_BR, 1), lambda i: (i, 0)),
        pl.BlockSpec((_D, _D), lambda i: (0, 0)),
        pl.BlockSpec((1, _D), lambda i: (0, 0)),
    ],
    out_specs=pl.BlockSpec((_BR, _D), lambda i: (i, 0)),
    out_shape=jax.ShapeDtypeStruct((_N, _D), _f32),
)


def kernel(feat, edge_index, W, b):
    pad_iota = jnp.arange(_EPAD, dtype=jnp.int32)
    pad_src = pad_iota % _N
    pad_dst = _N + (pad_iota % _PR)
    src = jnp.concatenate([edge_index[0], pad_src])
    dst = jnp.concatenate([edge_index[1], pad_dst]).reshape(_ER, _C)
    degp = _deg_kernel(dst)
    norm, g1 = _norm_call(degp.reshape(_NC, _N, 1), feat)
    p1 = _hop_kernel(g1, src, dst)
    g2 = _mid_call(p1, norm)
    p2 = _hop_kernel(g2, src, dst)
    out = _fc_call(p2, norm, W, b.reshape(1, _D))
    return out
